# Initial kernel scaffold; baseline (speedup 1.0000x reference)
import jax
import jax.numpy as jnp
from jax.experimental import pallas as pl


def kernel(preds, targets, masks, img_w, img_h):
    raise NotImplementedError("write your pallas kernel here")


# speedup vs baseline: 6.8136x; 6.8136x over previous
import functools

import numpy as np
import jax
import jax.numpy as jnp
from jax import lax
from jax.experimental import pallas as pl
from jax.experimental.pallas import tpu as pltpu


def _body(par_ref, sysc_ref, pr_ref, gsc_ref, gdel_ref, gcol_ref,
          outm_ref, outi_ref, cur_ref, vp_ref, rows_ref, iou_ref, cost_ref,
          wk_ref, sel_ref, ssum_ref, *, no, t_gt, p_pr):
    f32 = jnp.float32
    t = pl.program_id(1)
    h1 = par_ref[0:1, 0:1]
    w1 = par_ref[0:1, 1:2]
    hh = par_ref[0:1, 2:3]
    DEG = f32(np.pi / 180.0)
    INV = f32(-100000.0)
    sysc = sysc_ref[:, :]

    @pl.when(t == 0)
    def _setup():
        l0 = pr_ref[0, 0:1, :]
        l1 = pr_ref[0, 1:2, :]
        mx = jnp.maximum(l0, l1)
        e0 = jnp.exp(l0 - mx)
        e1 = jnp.exp(l1 - mx)
        s1 = e1 / (e0 + e1)
        cls4 = f32(-4.0) * jnp.log(jnp.maximum(s1, f32(1e-8)))

        sy = pr_ref[0, 2:3, :]
        sx = pr_ref[0, 3:4, :]
        th = pr_ref[0, 4:5, :]
        ln = pr_ref[0, 5:6, :]
        ptan = jnp.clip(jnp.tan(th * DEG), -1000.0, 1000.0)
        pxs = sx + (sy - sysc) * ptan + pr_ref[0, 6:6 + no, :]
        sidx = (f32(1.0) - sy / h1) * f32(no - 1)
        lidx = ln / hh * f32(no - 1)
        eio = lax.broadcasted_iota(jnp.int32, (no, 1), 0).astype(f32)
        pmask = (eio >= sidx) & (eio <= sidx + lidx)
        cur = jnp.where(pmask, pxs, INV)
        cur_ref[:, :] = cur
        vp_ref[:, :] = jnp.where(cur != INV, f32(1.0), f32(0.0))
        rows_ref[0:1, :] = cls4
        rows_ref[1:2, :] = sx / w1
        rows_ref[2:3, :] = sy / h1
        rows_ref[3:4, :] = th / f32(90.0)

    gsc = gsc_ref[0, 0]
    gsx = gsc[0:1, 0:1]
    gsy = gsc[0:1, 1:2]
    gth = gsc[0:1, 2:3]
    gmk = gsc[0:1, 3:4]
    gtan = jnp.clip(jnp.tan(gth * DEG), -1000.0, 1000.0)
    gdel = gdel_ref[0, 0]
    gxs = gsx + (gsy - sysc) * gtan + gdel
    gxs = jnp.where(gdel < -10000.0, INV, gxs)
    gvt = jnp.where(gxs != INV, f32(1.0), f32(0.0))

    m = vp_ref[:, :] * gvt
    d = jnp.abs(cur_ref[:, :] - gxs)
    sum_d = jnp.sum(m * d, axis=0, keepdims=True)
    n30 = f32(30.0) * jnp.sum(m, axis=0, keepdims=True)
    iou_row = (n30 - sum_d) / (n30 + sum_d + f32(1e-9))
    iou_row = jnp.where(gmk != f32(0.0), iou_row, f32(0.0))
    iou_ref[pl.ds(t, 1), :] = iou_row

    dist = jnp.sqrt((rows_ref[1:2, :] - gsx / w1) ** 2
                    + (rows_ref[2:3, :] - gsy / h1) ** 2 + f32(1e-8))
    geom = dist + f32(2.0) * jnp.abs(rows_ref[3:4, :] - gth / f32(90.0))
    cost_row = (rows_ref[0:1, :] + f32(5.0) * geom) \
        + f32(2.0) * (f32(1.0) - iou_row)
    cost_ref[pl.ds(t, 1), :] = cost_row

    @pl.when(t == t_gt - 1)
    def _select():
        cost = cost_ref[:, :]
        iou_all = iou_ref[:, :]
        mk_c = gcol_ref[0][:, 3:4] != f32(0.0)
        L2 = lax.broadcasted_iota(jnp.int32, (t_gt, p_pr), 1)
        BIGI = jnp.int32(2 ** 30)

        wk_ref[:, :] = iou_all
        ssum_ref[:, :] = jnp.zeros((t_gt, 1), f32)

        def bk(_, c):
            work = wk_ref[:, :]
            mv = jnp.max(work, axis=1, keepdims=True)
            ii = jnp.min(jnp.where(work == mv, L2, BIGI), axis=1,
                         keepdims=True)
            wk_ref[:, :] = jnp.where(L2 == ii, f32(-3e38), work)
            ssum_ref[:, :] = ssum_ref[:, :] + mv
            return c

        lax.fori_loop(0, 10, bk, 0)
        kk = jnp.clip(ssum_ref[:, :].astype(jnp.int32), 1, p_pr)

        wk_ref[:, :] = cost
        sel_ref[:, :] = jnp.zeros((t_gt, p_pr), jnp.int32)

        def bs(r, c):
            workc = wk_ref[:, :]
            mv = jnp.min(workc, axis=1, keepdims=True)
            ii = jnp.min(jnp.where(workc == mv, L2, BIGI), axis=1,
                         keepdims=True)
            hit = L2 == ii
            sel_ref[:, :] = sel_ref[:, :] | (hit & (r < kk)).astype(jnp.int32)
            wk_ref[:, :] = jnp.where(hit, f32(3e38), workc)
            return c

        lax.fori_loop(0, 10, bs, 0)
        sel = (sel_ref[:, :] != 0) & mk_c

        SENT = f32(100000000.0)
        A = jnp.where(sel, cost, SENT)
        best = jnp.min(A, axis=0, keepdims=True)
        S2 = lax.broadcasted_iota(jnp.int32, (t_gt, p_pr), 0)
        mt = jnp.min(jnp.where(A == best, S2, BIGI), axis=0, keepdims=True)
        assigned = best < SENT
        outm_ref[0, :, :] = assigned.astype(jnp.int32)
        outi_ref[0, :, :] = jnp.where(assigned, mt, jnp.int32(-1))


def _run(predsT, gt_scal, gdelC, gt_cols, sys_col, params, *, b, p, t, no,
         interpret=False):
    body = functools.partial(_body, no=no, t_gt=t, p_pr=p)
    return pl.pallas_call(
        body,
        grid=(b, t),
        in_specs=[
            pl.BlockSpec((1, 4), lambda i, j: (0, 0)),
            pl.BlockSpec((no, 1), lambda i, j: (0, 0)),
            pl.BlockSpec((1, 6 + no, p), lambda i, j: (i, 0, 0)),
            pl.BlockSpec((1, 1, 1, 4), lambda i, j: (i, j, 0, 0)),
            pl.BlockSpec((1, 1, no, 1), lambda i, j: (i, j, 0, 0)),
            pl.BlockSpec((1, t, 4), lambda i, j: (i, 0, 0)),
        ],
        out_specs=[
            pl.BlockSpec((1, 1, p), lambda i, j: (i, 0, 0)),
            pl.BlockSpec((1, 1, p), lambda i, j: (i, 0, 0)),
        ],
        out_shape=[
            jax.ShapeDtypeStruct((b, 1, p), jnp.int32),
            jax.ShapeDtypeStruct((b, 1, p), jnp.int32),
        ],
        scratch_shapes=[
            pltpu.VMEM((no, p), jnp.float32),
            pltpu.VMEM((no, p), jnp.float32),
            pltpu.VMEM((8, p), jnp.float32),
            pltpu.VMEM((t, p), jnp.float32),
            pltpu.VMEM((t, p), jnp.float32),
            pltpu.VMEM((t, p), jnp.float32),
            pltpu.VMEM((t, p), jnp.int32),
            pltpu.VMEM((t, 1), jnp.float32),
        ],
        compiler_params=pltpu.CompilerParams(
            dimension_semantics=("arbitrary", "arbitrary"),
        ),
        interpret=interpret,
    )(params, sys_col, predsT, gt_scal, gdelC, gt_cols)


def kernel(preds, targets, masks, img_w, img_h):
    b, p, d6 = preds.shape
    t = targets.shape[1]
    no = d6 - 6
    f32 = jnp.float32

    predsT = jnp.swapaxes(preds, 1, 2)
    mask_f = masks.astype(f32)
    gt_cols = jnp.stack(
        [targets[:, :, 3], targets[:, :, 2], targets[:, :, 4], mask_f],
        axis=2)
    gt_scal = gt_cols.reshape(b, t, 1, 4)
    gdelC = targets[:, :, 6:].reshape(b, t, no, 1)
    sys_col = (jnp.linspace(0.0, 1.0, no, dtype=f32)
               * (img_h - 1)).reshape(no, 1)
    params = jnp.stack([jnp.float32(img_h - 1), jnp.float32(img_w - 1),
                        jnp.float32(img_h), jnp.float32(0.0)]).reshape(1, 4)

    outm, outi = _run(predsT, gt_scal, gdelC, gt_cols, sys_col, params,
                      b=b, p=p, t=t, no=no)
    assigned = outm.reshape(b, p).astype(bool)
    matched = outi.reshape(b, p).astype(jax.dtypes.canonicalize_dtype(np.int64))
    return assigned, matched


# speedup vs baseline: 7.6682x; 1.1254x over previous
import functools

import numpy as np
import jax
import jax.numpy as jnp
from jax import lax
from jax.experimental import pallas as pl
from jax.experimental.pallas import tpu as pltpu


def _body(par_ref, sysc_ref, sysr_ref, pr_ref, gsc_ref, gdel_ref, gcol_ref,
          gdelw_ref, outm_ref, outi_ref, cur_ref, vp_ref, rows_ref, iou_ref,
          cost_ref, wk_ref, sel_ref, ssum_ref, n30_ref, *, no, t_gt, p_pr):
    f32 = jnp.float32
    t = pl.program_id(1)
    h1 = par_ref[0:1, 0:1]
    w1 = par_ref[0:1, 1:2]
    hh = par_ref[0:1, 2:3]
    DEG = f32(np.pi / 180.0)
    INV = f32(-100000.0)
    sysc = sysc_ref[:, :]

    @pl.when(t == 0)
    def _setup():
        l0 = pr_ref[0, 0:1, :]
        l1 = pr_ref[0, 1:2, :]
        mx = jnp.maximum(l0, l1)
        e0 = jnp.exp(l0 - mx)
        e1 = jnp.exp(l1 - mx)
        s1 = e1 / (e0 + e1)
        cls4 = f32(-4.0) * jnp.log(jnp.maximum(s1, f32(1e-8)))

        sy = pr_ref[0, 2:3, :]
        sx = pr_ref[0, 3:4, :]
        th = pr_ref[0, 4:5, :]
        ln = pr_ref[0, 5:6, :]
        ptan = jnp.clip(jnp.tan(th * DEG), -1000.0, 1000.0)
        pxs = sx + (sy - sysc) * ptan + pr_ref[0, 6:6 + no, :]
        sidx = (f32(1.0) - sy / h1) * f32(no - 1)
        lidx = ln / hh * f32(no - 1)
        eio = lax.broadcasted_iota(jnp.int32, (no, 1), 0).astype(f32)
        pmask = (eio >= sidx) & (eio <= sidx + lidx)
        cur = jnp.where(pmask, pxs, INV)
        cur_ref[:, :] = cur
        vp_ref[:, :] = jnp.where(cur != INV, f32(1.0), f32(0.0))
        rows_ref[0:1, :] = cls4
        rows_ref[1:2, :] = sx / w1
        rows_ref[2:3, :] = sy / h1
        rows_ref[3:4, :] = th / f32(90.0)

        gc0 = gcol_ref[0]
        gtanc = jnp.clip(jnp.tan(gc0[:, 2:3] * DEG), -1000.0, 1000.0)
        gxsw = gc0[:, 0:1] + (gc0[:, 1:2] - sysr_ref[:, :]) * gtanc \
            + gdelw_ref[0]
        gxsw = jnp.where(gdelw_ref[0] < -10000.0, INV, gxsw)
        gvw = jnp.where(gxsw != INV, f32(1.0), f32(0.0))
        n30_ref[:, :] = f32(30.0) * jax.lax.dot_general(
            gvw, vp_ref[:, :], (((1,), (0,)), ((), ())),
            preferred_element_type=jnp.float32)

    gsc = gsc_ref[0, 0]
    gsx = gsc[0:1, 0:1]
    gsy = gsc[0:1, 1:2]
    gth = gsc[0:1, 2:3]
    gmk = gsc[0:1, 3:4]
    gtan = jnp.clip(jnp.tan(gth * DEG), -1000.0, 1000.0)
    gdel = gdel_ref[0, 0]
    gxs = gsx + (gsy - sysc) * gtan + gdel
    gxs = jnp.where(gdel < -10000.0, INV, gxs)
    gvt = jnp.where(gxs != INV, f32(1.0), f32(0.0))

    m = vp_ref[:, :] * gvt
    d = jnp.abs(cur_ref[:, :] - gxs)
    sum_d = jnp.sum(m * d, axis=0, keepdims=True)
    n30 = n30_ref[pl.ds(t, 1), :]
    iou_row = (n30 - sum_d) / (n30 + sum_d + f32(1e-9))
    iou_row = jnp.where(gmk != f32(0.0), iou_row, f32(0.0))
    iou_ref[pl.ds(t, 1), :] = iou_row

    dist = jnp.sqrt((rows_ref[1:2, :] - gsx / w1) ** 2
                    + (rows_ref[2:3, :] - gsy / h1) ** 2 + f32(1e-8))
    geom = dist + f32(2.0) * jnp.abs(rows_ref[3:4, :] - gth / f32(90.0))
    cost_row = (rows_ref[0:1, :] + f32(5.0) * geom) \
        + f32(2.0) * (f32(1.0) - iou_row)
    cost_ref[pl.ds(t, 1), :] = cost_row

    @pl.when(t == t_gt - 1)
    def _select():
        cost = cost_ref[:, :]
        iou_all = iou_ref[:, :]
        mk_c = gcol_ref[0][:, 3:4] != f32(0.0)
        L2 = lax.broadcasted_iota(jnp.int32, (t_gt, p_pr), 1)
        BIGI = jnp.int32(2 ** 30)

        wk_ref[:, :] = iou_all
        ssum_ref[:, :] = jnp.zeros((t_gt, 1), f32)

        def bk(_, c):
            work = wk_ref[:, :]
            mv = jnp.max(work, axis=1, keepdims=True)
            ii = jnp.min(jnp.where(work == mv, L2, BIGI), axis=1,
                         keepdims=True)
            wk_ref[:, :] = jnp.where(L2 == ii, f32(-3e38), work)
            ssum_ref[:, :] = ssum_ref[:, :] + mv
            return c

        lax.fori_loop(0, 10, bk, 0)
        kk = jnp.clip(ssum_ref[:, :].astype(jnp.int32), 1, p_pr)

        wk_ref[:, :] = cost
        sel_ref[:, :] = jnp.zeros((t_gt, p_pr), jnp.int32)

        def bs(r, c):
            workc = wk_ref[:, :]
            mv = jnp.min(workc, axis=1, keepdims=True)
            ii = jnp.min(jnp.where(workc == mv, L2, BIGI), axis=1,
                         keepdims=True)
            hit = L2 == ii
            sel_ref[:, :] = sel_ref[:, :] | (hit & (r < kk)).astype(jnp.int32)
            wk_ref[:, :] = jnp.where(hit, f32(3e38), workc)
            return c

        lax.fori_loop(0, 10, bs, 0)
        sel = (sel_ref[:, :] != 0) & mk_c

        SENT = f32(100000000.0)
        A = jnp.where(sel, cost, SENT)
        best = jnp.min(A, axis=0, keepdims=True)
        S2 = lax.broadcasted_iota(jnp.int32, (t_gt, p_pr), 0)
        mt = jnp.min(jnp.where(A == best, S2, BIGI), axis=0, keepdims=True)
        assigned = best < SENT
        outm_ref[0, :, :] = assigned.astype(jnp.int32)
        outi_ref[0, :, :] = jnp.where(assigned, mt, jnp.int32(-1))


def _run(predsT, gt_scal, gdelC, gt_cols, gdelw, sys_col, sys_row, params,
         *, b, p, t, no, interpret=False):
    body = functools.partial(_body, no=no, t_gt=t, p_pr=p)
    return pl.pallas_call(
        body,
        grid=(b, t),
        in_specs=[
            pl.BlockSpec((1, 4), lambda i, j: (0, 0)),
            pl.BlockSpec((no, 1), lambda i, j: (0, 0)),
            pl.BlockSpec((1, no), lambda i, j: (0, 0)),
            pl.BlockSpec((1, 6 + no, p), lambda i, j: (i, 0, 0)),
            pl.BlockSpec((1, 1, 1, 4), lambda i, j: (i, j, 0, 0)),
            pl.BlockSpec((1, 1, no, 1), lambda i, j: (i, j, 0, 0)),
            pl.BlockSpec((1, t, 4), lambda i, j: (i, 0, 0)),
            pl.BlockSpec((1, t, no), lambda i, j: (i, 0, 0)),
        ],
        out_specs=[
            pl.BlockSpec((1, 1, p), lambda i, j: (i, 0, 0)),
            pl.BlockSpec((1, 1, p), lambda i, j: (i, 0, 0)),
        ],
        out_shape=[
            jax.ShapeDtypeStruct((b, 1, p), jnp.int32),
            jax.ShapeDtypeStruct((b, 1, p), jnp.int32),
        ],
        scratch_shapes=[
            pltpu.VMEM((no, p), jnp.float32),
            pltpu.VMEM((no, p), jnp.float32),
            pltpu.VMEM((8, p), jnp.float32),
            pltpu.VMEM((t, p), jnp.float32),
            pltpu.VMEM((t, p), jnp.float32),
            pltpu.VMEM((t, p), jnp.float32),
            pltpu.VMEM((t, p), jnp.int32),
            pltpu.VMEM((t, 1), jnp.float32),
            pltpu.VMEM((t, p), jnp.float32),
        ],
        compiler_params=pltpu.CompilerParams(
            dimension_semantics=("arbitrary", "arbitrary"),
        ),
        interpret=interpret,
    )(params, sys_col, sys_row, predsT, gt_scal, gdelC, gt_cols, gdelw)


def kernel(preds, targets, masks, img_w, img_h):
    b, p, d6 = preds.shape
    t = targets.shape[1]
    no = d6 - 6
    f32 = jnp.float32

    predsT = jnp.swapaxes(preds, 1, 2)
    mask_f = masks.astype(f32)
    gt_cols = jnp.stack(
        [targets[:, :, 3], targets[:, :, 2], targets[:, :, 4], mask_f],
        axis=2)
    gt_scal = gt_cols.reshape(b, t, 1, 4)
    gdelw = targets[:, :, 6:]
    gdelC = gdelw.reshape(b, t, no, 1)
    sys = jnp.linspace(0.0, 1.0, no, dtype=f32) * (img_h - 1)
    sys_col = sys.reshape(no, 1)
    sys_row = sys.reshape(1, no)
    params = jnp.stack([jnp.float32(img_h - 1), jnp.float32(img_w - 1),
                        jnp.float32(img_h), jnp.float32(0.0)]).reshape(1, 4)

    outm, outi = _run(predsT, gt_scal, gdelC, gt_cols, gdelw, sys_col,
                      sys_row, params, b=b, p=p, t=t, no=no)
    assigned = outm.reshape(b, p).astype(bool)
    matched = outi.reshape(b, p).astype(jax.dtypes.canonicalize_dtype(np.int64))
    return assigned, matched


# speedup vs baseline: 8.2273x; 1.0729x over previous
import functools

import numpy as np
import jax
import jax.numpy as jnp
from jax import lax
from jax.experimental import pallas as pl
from jax.experimental.pallas import tpu as pltpu


def _body(par_ref, sysc_ref, sysr_ref, pr_ref, gsc_ref, gdel_ref, gcol_ref,
          gdelw_ref, outm_ref, outi_ref, cur_ref, vp_ref, rows_ref, iou_ref,
          cost_ref, wk_ref, sel_ref, ssum_ref, n30_ref, *, no, t_gt, p_pr):
    f32 = jnp.float32
    t = pl.program_id(1)
    h1 = par_ref[0:1, 0:1]
    w1 = par_ref[0:1, 1:2]
    hh = par_ref[0:1, 2:3]
    DEG = f32(np.pi / 180.0)
    INV = f32(-100000.0)
    sysc = sysc_ref[:, :]

    @pl.when(t == 0)
    def _setup():
        l0 = pr_ref[0, 0:1, :]
        l1 = pr_ref[0, 1:2, :]
        mx = jnp.maximum(l0, l1)
        e0 = jnp.exp(l0 - mx)
        e1 = jnp.exp(l1 - mx)
        s1 = e1 / (e0 + e1)
        cls4 = f32(-4.0) * jnp.log(jnp.maximum(s1, f32(1e-8)))

        sy = pr_ref[0, 2:3, :]
        sx = pr_ref[0, 3:4, :]
        th = pr_ref[0, 4:5, :]
        ln = pr_ref[0, 5:6, :]
        ptan = jnp.clip(jnp.tan(th * DEG), -1000.0, 1000.0)
        pxs = sx + (sy - sysc) * ptan + pr_ref[0, 6:6 + no, :]
        sidx = (f32(1.0) - sy / h1) * f32(no - 1)
        lidx = ln / hh * f32(no - 1)
        eio = lax.broadcasted_iota(jnp.int32, (no, 1), 0).astype(f32)
        pmask = (eio >= sidx) & (eio <= sidx + lidx)
        cur = jnp.where(pmask, pxs, INV)
        cur_ref[:, :] = cur
        vp_ref[:, :] = jnp.where(cur != INV, f32(1.0), f32(0.0))
        rows_ref[0:1, :] = cls4
        rows_ref[1:2, :] = sx / w1
        rows_ref[2:3, :] = sy / h1
        rows_ref[3:4, :] = th / f32(90.0)

        gc0 = gcol_ref[0]
        gtanc = jnp.clip(jnp.tan(gc0[:, 2:3] * DEG), -1000.0, 1000.0)
        gxsw = gc0[:, 0:1] + (gc0[:, 1:2] - sysr_ref[:, :]) * gtanc \
            + gdelw_ref[0]
        gxsw = jnp.where(gdelw_ref[0] < -10000.0, INV, gxsw)
        gvw = jnp.where(gxsw != INV, f32(1.0), f32(0.0))
        n30_ref[:, :] = f32(30.0) * jax.lax.dot_general(
            gvw, vp_ref[:, :], (((1,), (0,)), ((), ())),
            preferred_element_type=jnp.float32)

    vp = vp_ref[:, :]
    cur = cur_ref[:, :]
    for jj in range(2):
        gsc = gsc_ref[0, jj]
        gsx = gsc[0:1, 0:1]
        gsy = gsc[0:1, 1:2]
        gth = gsc[0:1, 2:3]
        gmk = gsc[0:1, 3:4]
        gtan = jnp.clip(jnp.tan(gth * DEG), -1000.0, 1000.0)
        gdel = gdel_ref[0, jj]
        gxs = gsx + (gsy - sysc) * gtan + gdel
        gxs = jnp.where(gdel < -10000.0, INV, gxs)
        gvt = jnp.where(gxs != INV, f32(1.0), f32(0.0))

        m = vp * gvt
        d = jnp.abs(cur - gxs)
        sum_d = jnp.sum(m * d, axis=0, keepdims=True)
        n30 = n30_ref[pl.ds(2 * t + jj, 1), :]
        iou_row = (n30 - sum_d) / (n30 + sum_d + f32(1e-9))
        iou_row = jnp.where(gmk != f32(0.0), iou_row, f32(0.0))
        iou_ref[pl.ds(2 * t + jj, 1), :] = iou_row

        dist = jnp.sqrt((rows_ref[1:2, :] - gsx / w1) ** 2
                        + (rows_ref[2:3, :] - gsy / h1) ** 2 + f32(1e-8))
        geom = dist + f32(2.0) * jnp.abs(rows_ref[3:4, :] - gth / f32(90.0))
        cost_row = (rows_ref[0:1, :] + f32(5.0) * geom) \
            + f32(2.0) * (f32(1.0) - iou_row)
        cost_ref[pl.ds(2 * t + jj, 1), :] = cost_row

    @pl.when(t == t_gt // 2 - 1)
    def _select():
        cost = cost_ref[:, :]
        iou_all = iou_ref[:, :]
        mk_c = gcol_ref[0][:, 3:4] != f32(0.0)
        L2 = lax.broadcasted_iota(jnp.int32, (t_gt, p_pr), 1)
        BIGI = jnp.int32(2 ** 30)

        wk_ref[:, :] = iou_all
        ssum_ref[:, :] = jnp.zeros((t_gt, 1), f32)

        def bk(_, c):
            work = wk_ref[:, :]
            mv = jnp.max(work, axis=1, keepdims=True)
            ii = jnp.min(jnp.where(work == mv, L2, BIGI), axis=1,
                         keepdims=True)
            wk_ref[:, :] = jnp.where(L2 == ii, f32(-3e38), work)
            ssum_ref[:, :] = ssum_ref[:, :] + mv
            return c

        lax.fori_loop(0, 10, bk, 0)
        kk = jnp.clip(ssum_ref[:, :].astype(jnp.int32), 1, p_pr)

        wk_ref[:, :] = cost
        sel_ref[:, :] = jnp.zeros((t_gt, p_pr), jnp.int32)

        def bs(r, c):
            workc = wk_ref[:, :]
            mv = jnp.min(workc, axis=1, keepdims=True)
            ii = jnp.min(jnp.where(workc == mv, L2, BIGI), axis=1,
                         keepdims=True)
            hit = L2 == ii
            sel_ref[:, :] = sel_ref[:, :] | (hit & (r < kk)).astype(jnp.int32)
            wk_ref[:, :] = jnp.where(hit, f32(3e38), workc)
            return c

        lax.fori_loop(0, 10, bs, 0)
        sel = (sel_ref[:, :] != 0) & mk_c

        SENT = f32(100000000.0)
        A = jnp.where(sel, cost, SENT)
        best = jnp.min(A, axis=0, keepdims=True)
        S2 = lax.broadcasted_iota(jnp.int32, (t_gt, p_pr), 0)
        mt = jnp.min(jnp.where(A == best, S2, BIGI), axis=0, keepdims=True)
        assigned = best < SENT
        outm_ref[0, :, :] = assigned.astype(jnp.int32)
        outi_ref[0, :, :] = jnp.where(assigned, mt, jnp.int32(-1))


def _run(predsT, gt_scal, gdelC, gt_cols, gdelw, sys_col, sys_row, params,
         *, b, p, t, no, interpret=False):
    body = functools.partial(_body, no=no, t_gt=t, p_pr=p)
    return pl.pallas_call(
        body,
        grid=(b, t // 2),
        in_specs=[
            pl.BlockSpec((1, 4), lambda i, j: (0, 0)),
            pl.BlockSpec((no, 1), lambda i, j: (0, 0)),
            pl.BlockSpec((1, no), lambda i, j: (0, 0)),
            pl.BlockSpec((1, 6 + no, p), lambda i, j: (i, 0, 0)),
            pl.BlockSpec((1, 2, 1, 4), lambda i, j: (i, j, 0, 0)),
            pl.BlockSpec((1, 2, no, 1), lambda i, j: (i, j, 0, 0)),
            pl.BlockSpec((1, t, 4), lambda i, j: (i, 0, 0)),
            pl.BlockSpec((1, t, no), lambda i, j: (i, 0, 0)),
        ],
        out_specs=[
            pl.BlockSpec((1, 1, p), lambda i, j: (i, 0, 0)),
            pl.BlockSpec((1, 1, p), lambda i, j: (i, 0, 0)),
        ],
        out_shape=[
            jax.ShapeDtypeStruct((b, 1, p), jnp.int32),
            jax.ShapeDtypeStruct((b, 1, p), jnp.int32),
        ],
        scratch_shapes=[
            pltpu.VMEM((no, p), jnp.float32),
            pltpu.VMEM((no, p), jnp.float32),
            pltpu.VMEM((8, p), jnp.float32),
            pltpu.VMEM((t, p), jnp.float32),
            pltpu.VMEM((t, p), jnp.float32),
            pltpu.VMEM((t, p), jnp.float32),
            pltpu.VMEM((t, p), jnp.int32),
            pltpu.VMEM((t, 1), jnp.float32),
            pltpu.VMEM((t, p), jnp.float32),
        ],
        compiler_params=pltpu.CompilerParams(
            dimension_semantics=("arbitrary", "arbitrary"),
        ),
        interpret=interpret,
    )(params, sys_col, sys_row, predsT, gt_scal, gdelC, gt_cols, gdelw)


def kernel(preds, targets, masks, img_w, img_h):
    b, p, d6 = preds.shape
    t = targets.shape[1]
    no = d6 - 6
    f32 = jnp.float32

    predsT = jnp.swapaxes(preds, 1, 2)
    mask_f = masks.astype(f32)
    gt_cols = jnp.stack(
        [targets[:, :, 3], targets[:, :, 2], targets[:, :, 4], mask_f],
        axis=2)
    gt_scal = gt_cols.reshape(b, t, 1, 4)
    gdelw = targets[:, :, 6:]
    gdelC = gdelw.reshape(b, t, no, 1)
    sys = jnp.linspace(0.0, 1.0, no, dtype=f32) * (img_h - 1)
    sys_col = sys.reshape(no, 1)
    sys_row = sys.reshape(1, no)
    params = jnp.stack([jnp.float32(img_h - 1), jnp.float32(img_w - 1),
                        jnp.float32(img_h), jnp.float32(0.0)]).reshape(1, 4)

    outm, outi = _run(predsT, gt_scal, gdelC, gt_cols, gdelw, sys_col,
                      sys_row, params, b=b, p=p, t=t, no=no)
    assigned = outm.reshape(b, p).astype(bool)
    matched = outi.reshape(b, p).astype(jax.dtypes.canonicalize_dtype(np.int64))
    return assigned, matched


# speedup vs baseline: 8.5778x; 1.0426x over previous
import functools

import numpy as np
import jax
import jax.numpy as jnp
from jax import lax
from jax.experimental import pallas as pl
from jax.experimental.pallas import tpu as pltpu


def _body(par_ref, sysc_ref, sysr_ref, pr_ref, gsc_ref, gdel_ref, gcol_ref,
          gdelw_ref, outm_ref, outi_ref, cur_ref, vp_ref, rows_ref, iou_ref,
          cost_ref, wk_ref, sel_ref, ssum_ref, n30_ref, *, no, t_gt, p_pr):
    f32 = jnp.float32
    t = pl.program_id(1)
    h1 = par_ref[0:1, 0:1]
    w1 = par_ref[0:1, 1:2]
    hh = par_ref[0:1, 2:3]
    DEG = f32(np.pi / 180.0)
    INV = f32(-100000.0)
    sysc = sysc_ref[:, :]

    @pl.when(t == 0)
    def _setup():
        l0 = pr_ref[0, 0:1, :]
        l1 = pr_ref[0, 1:2, :]
        mx = jnp.maximum(l0, l1)
        e0 = jnp.exp(l0 - mx)
        e1 = jnp.exp(l1 - mx)
        s1 = e1 / (e0 + e1)
        cls4 = f32(-4.0) * jnp.log(jnp.maximum(s1, f32(1e-8)))

        sy = pr_ref[0, 2:3, :]
        sx = pr_ref[0, 3:4, :]
        th = pr_ref[0, 4:5, :]
        ln = pr_ref[0, 5:6, :]
        ptan = jnp.clip(jnp.tan(th * DEG), -1000.0, 1000.0)
        pxs = sx + (sy - sysc) * ptan + pr_ref[0, 6:6 + no, :]
        sidx = (f32(1.0) - sy / h1) * f32(no - 1)
        lidx = ln / hh * f32(no - 1)
        eio = lax.broadcasted_iota(jnp.int32, (no, 1), 0).astype(f32)
        pmask = (eio >= sidx) & (eio <= sidx + lidx)
        cur = jnp.where(pmask, pxs, INV)
        cur_ref[:, :] = cur
        vp_ref[:, :] = jnp.where(cur != INV, f32(1.0), f32(0.0))
        rows_ref[0:1, :] = cls4
        rows_ref[1:2, :] = sx / w1
        rows_ref[2:3, :] = sy / h1
        rows_ref[3:4, :] = th / f32(90.0)

        gc0 = gcol_ref[0]
        gtanc = jnp.clip(jnp.tan(gc0[:, 2:3] * DEG), -1000.0, 1000.0)
        gxsw = gc0[:, 0:1] + (gc0[:, 1:2] - sysr_ref[:, :]) * gtanc \
            + gdelw_ref[0]
        gxsw = jnp.where(gdelw_ref[0] < -10000.0, INV, gxsw)
        gvw = jnp.where(gxsw != INV, f32(1.0), f32(0.0))
        n30_ref[:, :] = f32(30.0) * jax.lax.dot_general(
            gvw, vp_ref[:, :], (((1,), (0,)), ((), ())),
            preferred_element_type=jnp.float32)

    vp = vp_ref[:, :]
    cur = cur_ref[:, :]
    for jj in range(4):
        gsc = gsc_ref[0, jj]
        gsx = gsc[0:1, 0:1]
        gsy = gsc[0:1, 1:2]
        gth = gsc[0:1, 2:3]
        gmk = gsc[0:1, 3:4]
        gtan = jnp.clip(jnp.tan(gth * DEG), -1000.0, 1000.0)
        gdel = gdel_ref[0, jj]
        gxs = gsx + (gsy - sysc) * gtan + gdel
        gxs = jnp.where(gdel < -10000.0, INV, gxs)
        gvt = jnp.where(gxs != INV, f32(1.0), f32(0.0))

        m = vp * gvt
        d = jnp.abs(cur - gxs)
        sum_d = jnp.sum(m * d, axis=0, keepdims=True)
        n30 = n30_ref[pl.ds(4 * t + jj, 1), :]
        iou_row = (n30 - sum_d) / (n30 + sum_d + f32(1e-9))
        iou_row = jnp.where(gmk != f32(0.0), iou_row, f32(0.0))
        iou_ref[pl.ds(4 * t + jj, 1), :] = iou_row

        dist = jnp.sqrt((rows_ref[1:2, :] - gsx / w1) ** 2
                        + (rows_ref[2:3, :] - gsy / h1) ** 2 + f32(1e-8))
        geom = dist + f32(2.0) * jnp.abs(rows_ref[3:4, :] - gth / f32(90.0))
        cost_row = (rows_ref[0:1, :] + f32(5.0) * geom) \
            + f32(2.0) * (f32(1.0) - iou_row)
        cost_ref[pl.ds(4 * t + jj, 1), :] = cost_row

    @pl.when(t == t_gt // 4 - 1)
    def _select():
        cost = cost_ref[:, :]
        iou_all = iou_ref[:, :]
        mk_c = gcol_ref[0][:, 3:4] != f32(0.0)
        L2 = lax.broadcasted_iota(jnp.int32, (t_gt, p_pr), 1)
        BIGI = jnp.int32(2 ** 30)

        wk_ref[:, :] = iou_all
        ssum_ref[:, :] = jnp.zeros((t_gt, 1), f32)

        def bk(_, c):
            work = wk_ref[:, :]
            mv = jnp.max(work, axis=1, keepdims=True)
            ii = jnp.min(jnp.where(work == mv, L2, BIGI), axis=1,
                         keepdims=True)
            wk_ref[:, :] = jnp.where(L2 == ii, f32(-3e38), work)
            ssum_ref[:, :] = ssum_ref[:, :] + mv
            return c

        lax.fori_loop(0, 10, bk, 0)
        kk = jnp.clip(ssum_ref[:, :].astype(jnp.int32), 1, p_pr)

        wk_ref[:, :] = cost
        sel_ref[:, :] = jnp.zeros((t_gt, p_pr), jnp.int32)

        def bs(r, c):
            workc = wk_ref[:, :]
            mv = jnp.min(workc, axis=1, keepdims=True)
            ii = jnp.min(jnp.where(workc == mv, L2, BIGI), axis=1,
                         keepdims=True)
            hit = L2 == ii
            sel_ref[:, :] = sel_ref[:, :] | (hit & (r < kk)).astype(jnp.int32)
            wk_ref[:, :] = jnp.where(hit, f32(3e38), workc)
            return c

        lax.fori_loop(0, 10, bs, 0)
        sel = (sel_ref[:, :] != 0) & mk_c

        SENT = f32(100000000.0)
        A = jnp.where(sel, cost, SENT)
        best = jnp.min(A, axis=0, keepdims=True)
        S2 = lax.broadcasted_iota(jnp.int32, (t_gt, p_pr), 0)
        mt = jnp.min(jnp.where(A == best, S2, BIGI), axis=0, keepdims=True)
        assigned = best < SENT
        outm_ref[0, :, :] = assigned.astype(jnp.int32)
        outi_ref[0, :, :] = jnp.where(assigned, mt, jnp.int32(-1))


def _run(predsT, gt_scal, gdelC, gt_cols, gdelw, sys_col, sys_row, params,
         *, b, p, t, no, interpret=False):
    body = functools.partial(_body, no=no, t_gt=t, p_pr=p)
    return pl.pallas_call(
        body,
        grid=(b, t // 4),
        in_specs=[
            pl.BlockSpec((1, 4), lambda i, j: (0, 0)),
            pl.BlockSpec((no, 1), lambda i, j: (0, 0)),
            pl.BlockSpec((1, no), lambda i, j: (0, 0)),
            pl.BlockSpec((1, 6 + no, p), lambda i, j: (i, 0, 0)),
            pl.BlockSpec((1, 4, 1, 4), lambda i, j: (i, j, 0, 0)),
            pl.BlockSpec((1, 4, no, 1), lambda i, j: (i, j, 0, 0)),
            pl.BlockSpec((1, t, 4), lambda i, j: (i, 0, 0)),
            pl.BlockSpec((1, t, no), lambda i, j: (i, 0, 0)),
        ],
        out_specs=[
            pl.BlockSpec((1, 1, p), lambda i, j: (i, 0, 0)),
            pl.BlockSpec((1, 1, p), lambda i, j: (i, 0, 0)),
        ],
        out_shape=[
            jax.ShapeDtypeStruct((b, 1, p), jnp.int32),
            jax.ShapeDtypeStruct((b, 1, p), jnp.int32),
        ],
        scratch_shapes=[
            pltpu.VMEM((no, p), jnp.float32),
            pltpu.VMEM((no, p), jnp.float32),
            pltpu.VMEM((8, p), jnp.float32),
            pltpu.VMEM((t, p), jnp.float32),
            pltpu.VMEM((t, p), jnp.float32),
            pltpu.VMEM((t, p), jnp.float32),
            pltpu.VMEM((t, p), jnp.int32),
            pltpu.VMEM((t, 1), jnp.float32),
            pltpu.VMEM((t, p), jnp.float32),
        ],
        compiler_params=pltpu.CompilerParams(
            dimension_semantics=("arbitrary", "arbitrary"),
        ),
        interpret=interpret,
    )(params, sys_col, sys_row, predsT, gt_scal, gdelC, gt_cols, gdelw)


def kernel(preds, targets, masks, img_w, img_h):
    b, p, d6 = preds.shape
    t = targets.shape[1]
    no = d6 - 6
    f32 = jnp.float32

    predsT = jnp.swapaxes(preds, 1, 2)
    mask_f = masks.astype(f32)
    gt_cols = jnp.stack(
        [targets[:, :, 3], targets[:, :, 2], targets[:, :, 4], mask_f],
        axis=2)
    gt_scal = gt_cols.reshape(b, t, 1, 4)
    gdelw = targets[:, :, 6:]
    gdelC = gdelw.reshape(b, t, no, 1)
    sys = jnp.linspace(0.0, 1.0, no, dtype=f32) * (img_h - 1)
    sys_col = sys.reshape(no, 1)
    sys_row = sys.reshape(1, no)
    params = jnp.stack([jnp.float32(img_h - 1), jnp.float32(img_w - 1),
                        jnp.float32(img_h), jnp.float32(0.0)]).reshape(1, 4)

    outm, outi = _run(predsT, gt_scal, gdelC, gt_cols, gdelw, sys_col,
                      sys_row, params, b=b, p=p, t=t, no=no)
    assigned = outm.reshape(b, p).astype(bool)
    matched = outi.reshape(b, p).astype(jax.dtypes.canonicalize_dtype(np.int64))
    return assigned, matched


# speedup vs baseline: 8.7467x; 1.0197x over previous
import functools

import numpy as np
import jax
import jax.numpy as jnp
from jax import lax
from jax.experimental import pallas as pl
from jax.experimental.pallas import tpu as pltpu


def _body(par_ref, sysc_ref, sysr_ref, pr_ref, gsc_ref, gdel_ref, gcol_ref,
          gdelw_ref, outm_ref, outi_ref, cur_ref, vp_ref, rows_ref, iou_ref,
          cost_ref, wk_ref, sel_ref, ssum_ref, n30_ref, *, no, t_gt, p_pr):
    f32 = jnp.float32
    t = pl.program_id(1)
    h1 = par_ref[0:1, 0:1]
    w1 = par_ref[0:1, 1:2]
    hh = par_ref[0:1, 2:3]
    DEG = f32(np.pi / 180.0)
    INV = f32(-100000.0)
    sysc = sysc_ref[:, :]

    @pl.when(t == 0)
    def _setup():
        l0 = pr_ref[0, 0:1, :]
        l1 = pr_ref[0, 1:2, :]
        mx = jnp.maximum(l0, l1)
        e0 = jnp.exp(l0 - mx)
        e1 = jnp.exp(l1 - mx)
        s1 = e1 / (e0 + e1)
        cls4 = f32(-4.0) * jnp.log(jnp.maximum(s1, f32(1e-8)))

        sy = pr_ref[0, 2:3, :]
        sx = pr_ref[0, 3:4, :]
        th = pr_ref[0, 4:5, :]
        ln = pr_ref[0, 5:6, :]
        ptan = jnp.clip(jnp.tan(th * DEG), -1000.0, 1000.0)
        pxs = sx + (sy - sysc) * ptan + pr_ref[0, 6:6 + no, :]
        sidx = (f32(1.0) - sy / h1) * f32(no - 1)
        lidx = ln / hh * f32(no - 1)
        eio = lax.broadcasted_iota(jnp.int32, (no, 1), 0).astype(f32)
        pmask = (eio >= sidx) & (eio <= sidx + lidx)
        cur = jnp.where(pmask, pxs, INV)
        cur_ref[:, :] = cur
        vp_ref[:, :] = jnp.where(cur != INV, f32(1.0), f32(0.0))
        rows_ref[0:1, :] = cls4
        rows_ref[1:2, :] = sx / w1
        rows_ref[2:3, :] = sy / h1
        rows_ref[3:4, :] = th / f32(90.0)

        gc0 = gcol_ref[0]
        gtanc = jnp.clip(jnp.tan(gc0[:, 2:3] * DEG), -1000.0, 1000.0)
        gxsw = gc0[:, 0:1] + (gc0[:, 1:2] - sysr_ref[:, :]) * gtanc \
            + gdelw_ref[0]
        gxsw = jnp.where(gdelw_ref[0] < -10000.0, INV, gxsw)
        gvw = jnp.where(gxsw != INV, f32(1.0), f32(0.0))
        n30_ref[:, :] = f32(30.0) * jax.lax.dot_general(
            gvw, vp_ref[:, :], (((1,), (0,)), ((), ())),
            preferred_element_type=jnp.float32)

    vp = vp_ref[:, :]
    cur = cur_ref[:, :]
    for jj in range(8):
        gsc = gsc_ref[0, jj]
        gsx = gsc[0:1, 0:1]
        gsy = gsc[0:1, 1:2]
        gth = gsc[0:1, 2:3]
        gmk = gsc[0:1, 3:4]
        gtan = jnp.clip(jnp.tan(gth * DEG), -1000.0, 1000.0)
        gdel = gdel_ref[0, jj]
        gxs = gsx + (gsy - sysc) * gtan + gdel
        gxs = jnp.where(gdel < -10000.0, INV, gxs)
        gvt = jnp.where(gxs != INV, f32(1.0), f32(0.0))

        m = vp * gvt
        d = jnp.abs(cur - gxs)
        sum_d = jnp.sum(m * d, axis=0, keepdims=True)
        n30 = n30_ref[pl.ds(8 * t + jj, 1), :]
        iou_row = (n30 - sum_d) / (n30 + sum_d + f32(1e-9))
        iou_row = jnp.where(gmk != f32(0.0), iou_row, f32(0.0))
        iou_ref[pl.ds(8 * t + jj, 1), :] = iou_row

        dist = jnp.sqrt((rows_ref[1:2, :] - gsx / w1) ** 2
                        + (rows_ref[2:3, :] - gsy / h1) ** 2 + f32(1e-8))
        geom = dist + f32(2.0) * jnp.abs(rows_ref[3:4, :] - gth / f32(90.0))
        cost_row = (rows_ref[0:1, :] + f32(5.0) * geom) \
            + f32(2.0) * (f32(1.0) - iou_row)
        cost_ref[pl.ds(8 * t + jj, 1), :] = cost_row

    @pl.when(t == t_gt // 8 - 1)
    def _select():
        cost = cost_ref[:, :]
        iou_all = iou_ref[:, :]
        mk_c = gcol_ref[0][:, 3:4] != f32(0.0)
        L2 = lax.broadcasted_iota(jnp.int32, (t_gt, p_pr), 1)
        BIGI = jnp.int32(2 ** 30)

        wk_ref[:, :] = iou_all
        ssum_ref[:, :] = jnp.zeros((t_gt, 1), f32)

        def bk(_, c):
            work = wk_ref[:, :]
            mv = jnp.max(work, axis=1, keepdims=True)
            ii = jnp.min(jnp.where(work == mv, L2, BIGI), axis=1,
                         keepdims=True)
            wk_ref[:, :] = jnp.where(L2 == ii, f32(-3e38), work)
            ssum_ref[:, :] = ssum_ref[:, :] + mv
            return c

        lax.fori_loop(0, 10, bk, 0)
        kk = jnp.clip(ssum_ref[:, :].astype(jnp.int32), 1, p_pr)

        wk_ref[:, :] = cost
        sel_ref[:, :] = jnp.zeros((t_gt, p_pr), jnp.int32)

        def bs(r, c):
            workc = wk_ref[:, :]
            mv = jnp.min(workc, axis=1, keepdims=True)
            ii = jnp.min(jnp.where(workc == mv, L2, BIGI), axis=1,
                         keepdims=True)
            hit = L2 == ii
            sel_ref[:, :] = sel_ref[:, :] | (hit & (r < kk)).astype(jnp.int32)
            wk_ref[:, :] = jnp.where(hit, f32(3e38), workc)
            return c

        lax.fori_loop(0, 10, bs, 0)
        sel = (sel_ref[:, :] != 0) & mk_c

        SENT = f32(100000000.0)
        A = jnp.where(sel, cost, SENT)
        best = jnp.min(A, axis=0, keepdims=True)
        S2 = lax.broadcasted_iota(jnp.int32, (t_gt, p_pr), 0)
        mt = jnp.min(jnp.where(A == best, S2, BIGI), axis=0, keepdims=True)
        assigned = best < SENT
        outm_ref[0, :, :] = assigned.astype(jnp.int32)
        outi_ref[0, :, :] = jnp.where(assigned, mt, jnp.int32(-1))


def _run(predsT, gt_scal, gdelC, gt_cols, gdelw, sys_col, sys_row, params,
         *, b, p, t, no, interpret=False):
    body = functools.partial(_body, no=no, t_gt=t, p_pr=p)
    return pl.pallas_call(
        body,
        grid=(b, t // 8),
        in_specs=[
            pl.BlockSpec((1, 4), lambda i, j: (0, 0)),
            pl.BlockSpec((no, 1), lambda i, j: (0, 0)),
            pl.BlockSpec((1, no), lambda i, j: (0, 0)),
            pl.BlockSpec((1, 6 + no, p), lambda i, j: (i, 0, 0)),
            pl.BlockSpec((1, 8, 1, 4), lambda i, j: (i, j, 0, 0)),
            pl.BlockSpec((1, 8, no, 1), lambda i, j: (i, j, 0, 0)),
            pl.BlockSpec((1, t, 4), lambda i, j: (i, 0, 0)),
            pl.BlockSpec((1, t, no), lambda i, j: (i, 0, 0)),
        ],
        out_specs=[
            pl.BlockSpec((1, 1, p), lambda i, j: (i, 0, 0)),
            pl.BlockSpec((1, 1, p), lambda i, j: (i, 0, 0)),
        ],
        out_shape=[
            jax.ShapeDtypeStruct((b, 1, p), jnp.int32),
            jax.ShapeDtypeStruct((b, 1, p), jnp.int32),
        ],
        scratch_shapes=[
            pltpu.VMEM((no, p), jnp.float32),
            pltpu.VMEM((no, p), jnp.float32),
            pltpu.VMEM((8, p), jnp.float32),
            pltpu.VMEM((t, p), jnp.float32),
            pltpu.VMEM((t, p), jnp.float32),
            pltpu.VMEM((t, p), jnp.float32),
            pltpu.VMEM((t, p), jnp.int32),
            pltpu.VMEM((t, 1), jnp.float32),
            pltpu.VMEM((t, p), jnp.float32),
        ],
        compiler_params=pltpu.CompilerParams(
            dimension_semantics=("arbitrary", "arbitrary"),
        ),
        interpret=interpret,
    )(params, sys_col, sys_row, predsT, gt_scal, gdelC, gt_cols, gdelw)


def kernel(preds, targets, masks, img_w, img_h):
    b, p, d6 = preds.shape
    t = targets.shape[1]
    no = d6 - 6
    f32 = jnp.float32

    predsT = jnp.swapaxes(preds, 1, 2)
    mask_f = masks.astype(f32)
    gt_cols = jnp.stack(
        [targets[:, :, 3], targets[:, :, 2], targets[:, :, 4], mask_f],
        axis=2)
    gt_scal = gt_cols.reshape(b, t, 1, 4)
    gdelw = targets[:, :, 6:]
    gdelC = gdelw.reshape(b, t, no, 1)
    sys = jnp.linspace(0.0, 1.0, no, dtype=f32) * (img_h - 1)
    sys_col = sys.reshape(no, 1)
    sys_row = sys.reshape(1, no)
    params = jnp.stack([jnp.float32(img_h - 1), jnp.float32(img_w - 1),
                        jnp.float32(img_h), jnp.float32(0.0)]).reshape(1, 4)

    outm, outi = _run(predsT, gt_scal, gdelC, gt_cols, gdelw, sys_col,
                      sys_row, params, b=b, p=p, t=t, no=no)
    assigned = outm.reshape(b, p).astype(bool)
    matched = outi.reshape(b, p).astype(jax.dtypes.canonicalize_dtype(np.int64))
    return assigned, matched


# speedup vs baseline: 12.7618x; 1.4590x over previous
import functools

import numpy as np
import jax
import jax.numpy as jnp
from jax import lax
from jax.experimental import pallas as pl
from jax.experimental.pallas import tpu as pltpu


def _body(par_ref, sysc_ref, sysr_ref, pr_ref, gsc_ref, gdel_ref, gcol_ref,
          gdelw_ref, outm_ref, outi_ref, cur_ref, vp_ref, rows_ref, iou_ref,
          cost_ref, wk_ref, sel_ref, ssum_ref, n30_ref, *, no, t_gt, p_pr):
    f32 = jnp.float32
    t = pl.program_id(1)
    h1 = par_ref[0:1, 0:1]
    w1 = par_ref[0:1, 1:2]
    hh = par_ref[0:1, 2:3]
    DEG = f32(np.pi / 180.0)
    INV = f32(-100000.0)
    sysc = sysc_ref[:, :]

    @pl.when(t == 0)
    def _setup():
        l0 = pr_ref[0, 0:1, :]
        l1 = pr_ref[0, 1:2, :]
        mx = jnp.maximum(l0, l1)
        e0 = jnp.exp(l0 - mx)
        e1 = jnp.exp(l1 - mx)
        s1 = e1 / (e0 + e1)
        cls4 = f32(-4.0) * jnp.log(jnp.maximum(s1, f32(1e-8)))

        sy = pr_ref[0, 2:3, :]
        sx = pr_ref[0, 3:4, :]
        th = pr_ref[0, 4:5, :]
        ln = pr_ref[0, 5:6, :]
        ptan = jnp.clip(jnp.tan(th * DEG), -1000.0, 1000.0)
        pxs = sx + (sy - sysc) * ptan + pr_ref[0, 6:6 + no, :]
        sidx = (f32(1.0) - sy / h1) * f32(no - 1)
        lidx = ln / hh * f32(no - 1)
        eio = lax.broadcasted_iota(jnp.int32, (no, 1), 0).astype(f32)
        pmask = (eio >= sidx) & (eio <= sidx + lidx)
        cur = jnp.where(pmask, pxs, INV)
        cur_ref[:, :] = cur
        vp_ref[:, :] = jnp.where(cur != INV, f32(1.0), f32(0.0))
        rows_ref[0:1, :] = cls4
        rows_ref[1:2, :] = sx / w1
        rows_ref[2:3, :] = sy / h1
        rows_ref[3:4, :] = th / f32(90.0)

        gc0 = gcol_ref[0]
        gtanc = jnp.clip(jnp.tan(gc0[:, 2:3] * DEG), -1000.0, 1000.0)
        gxsw = gc0[:, 0:1] + (gc0[:, 1:2] - sysr_ref[:, :]) * gtanc \
            + gdelw_ref[0]
        gxsw = jnp.where(gdelw_ref[0] < -10000.0, INV, gxsw)
        gvw = jnp.where(gxsw != INV, f32(1.0), f32(0.0))
        n30_ref[:, :] = f32(30.0) * jax.lax.dot_general(
            gvw, vp_ref[:, :], (((1,), (0,)), ((), ())),
            preferred_element_type=jnp.float32)

    vp = vp_ref[:, :]
    cur = cur_ref[:, :]
    ones_row = jnp.ones((1, no), f32)
    for jj in range(8):
        gsc = gsc_ref[0, jj]
        gsx = gsc[0:1, 0:1]
        gsy = gsc[0:1, 1:2]
        gth = gsc[0:1, 2:3]
        gmk = gsc[0:1, 3:4]
        gtan = jnp.clip(jnp.tan(gth * DEG), -1000.0, 1000.0)
        gdel = gdel_ref[0, jj]
        gxs = gsx + (gsy - sysc) * gtan + gdel
        gxs = jnp.where(gdel < -10000.0, INV, gxs)
        gvt = jnp.where(gxs != INV, f32(1.0), f32(0.0))

        m = vp * gvt
        d = jnp.abs(cur - gxs)
        sum_d = lax.dot_general(ones_row, m * d, (((1,), (0,)), ((), ())),
                                preferred_element_type=jnp.float32)
        n30 = n30_ref[pl.ds(8 * t + jj, 1), :]
        iou_row = (n30 - sum_d) / (n30 + sum_d + f32(1e-9))
        iou_row = jnp.where(gmk != f32(0.0), iou_row, f32(0.0))
        iou_ref[pl.ds(8 * t + jj, 1), :] = iou_row

        dist = jnp.sqrt((rows_ref[1:2, :] - gsx / w1) ** 2
                        + (rows_ref[2:3, :] - gsy / h1) ** 2 + f32(1e-8))
        geom = dist + f32(2.0) * jnp.abs(rows_ref[3:4, :] - gth / f32(90.0))
        cost_row = (rows_ref[0:1, :] + f32(5.0) * geom) \
            + f32(2.0) * (f32(1.0) - iou_row)
        cost_ref[pl.ds(8 * t + jj, 1), :] = cost_row

    @pl.when(t == t_gt // 8 - 1)
    def _select():
        cost = cost_ref[:, :]
        iou_all = iou_ref[:, :]
        mk_c = gcol_ref[0][:, 3:4] != f32(0.0)
        L2 = lax.broadcasted_iota(jnp.int32, (t_gt, p_pr), 1)
        BIGI = jnp.int32(2 ** 30)

        wk_ref[:, :] = iou_all
        ssum_ref[:, :] = jnp.zeros((t_gt, 1), f32)

        def bk(_, c):
            work = wk_ref[:, :]
            mv = jnp.max(work, axis=1, keepdims=True)
            ii = jnp.min(jnp.where(work == mv, L2, BIGI), axis=1,
                         keepdims=True)
            wk_ref[:, :] = jnp.where(L2 == ii, f32(-3e38), work)
            ssum_ref[:, :] = ssum_ref[:, :] + mv
            return c

        lax.fori_loop(0, 10, bk, 0)
        kk = jnp.clip(ssum_ref[:, :].astype(jnp.int32), 1, p_pr)

        wk_ref[:, :] = cost
        sel_ref[:, :] = jnp.zeros((t_gt, p_pr), jnp.int32)

        def bs(r, c):
            workc = wk_ref[:, :]
            mv = jnp.min(workc, axis=1, keepdims=True)
            ii = jnp.min(jnp.where(workc == mv, L2, BIGI), axis=1,
                         keepdims=True)
            hit = L2 == ii
            sel_ref[:, :] = sel_ref[:, :] | (hit & (r < kk)).astype(jnp.int32)
            wk_ref[:, :] = jnp.where(hit, f32(3e38), workc)
            return c

        lax.fori_loop(0, 10, bs, 0)
        sel = (sel_ref[:, :] != 0) & mk_c

        SENT = f32(100000000.0)
        A = jnp.where(sel, cost, SENT)
        best = jnp.min(A, axis=0, keepdims=True)
        S2 = lax.broadcasted_iota(jnp.int32, (t_gt, p_pr), 0)
        mt = jnp.min(jnp.where(A == best, S2, BIGI), axis=0, keepdims=True)
        assigned = best < SENT
        outm_ref[0, :, :] = assigned.astype(jnp.int32)
        outi_ref[0, :, :] = jnp.where(assigned, mt, jnp.int32(-1))


def _run(predsT, gt_scal, gdelC, gt_cols, gdelw, sys_col, sys_row, params,
         *, b, p, t, no, interpret=False):
    body = functools.partial(_body, no=no, t_gt=t, p_pr=p)
    return pl.pallas_call(
        body,
        grid=(b, t // 8),
        in_specs=[
            pl.BlockSpec((1, 4), lambda i, j: (0, 0)),
            pl.BlockSpec((no, 1), lambda i, j: (0, 0)),
            pl.BlockSpec((1, no), lambda i, j: (0, 0)),
            pl.BlockSpec((1, 6 + no, p), lambda i, j: (i, 0, 0)),
            pl.BlockSpec((1, 8, 1, 4), lambda i, j: (i, j, 0, 0)),
            pl.BlockSpec((1, 8, no, 1), lambda i, j: (i, j, 0, 0)),
            pl.BlockSpec((1, t, 4), lambda i, j: (i, 0, 0)),
            pl.BlockSpec((1, t, no), lambda i, j: (i, 0, 0)),
        ],
        out_specs=[
            pl.BlockSpec((1, 1, p), lambda i, j: (i, 0, 0)),
            pl.BlockSpec((1, 1, p), lambda i, j: (i, 0, 0)),
        ],
        out_shape=[
            jax.ShapeDtypeStruct((b, 1, p), jnp.int32),
            jax.ShapeDtypeStruct((b, 1, p), jnp.int32),
        ],
        scratch_shapes=[
            pltpu.VMEM((no, p), jnp.float32),
            pltpu.VMEM((no, p), jnp.float32),
            pltpu.VMEM((8, p), jnp.float32),
            pltpu.VMEM((t, p), jnp.float32),
            pltpu.VMEM((t, p), jnp.float32),
            pltpu.VMEM((t, p), jnp.float32),
            pltpu.VMEM((t, p), jnp.int32),
            pltpu.VMEM((t, 1), jnp.float32),
            pltpu.VMEM((t, p), jnp.float32),
        ],
        compiler_params=pltpu.CompilerParams(
            dimension_semantics=("arbitrary", "arbitrary"),
        ),
        interpret=interpret,
    )(params, sys_col, sys_row, predsT, gt_scal, gdelC, gt_cols, gdelw)


def kernel(preds, targets, masks, img_w, img_h):
    b, p, d6 = preds.shape
    t = targets.shape[1]
    no = d6 - 6
    f32 = jnp.float32

    predsT = jnp.swapaxes(preds, 1, 2)
    mask_f = masks.astype(f32)
    gt_cols = jnp.stack(
        [targets[:, :, 3], targets[:, :, 2], targets[:, :, 4], mask_f],
        axis=2)
    gt_scal = gt_cols.reshape(b, t, 1, 4)
    gdelw = targets[:, :, 6:]
    gdelC = gdelw.reshape(b, t, no, 1)
    sys = jnp.linspace(0.0, 1.0, no, dtype=f32) * (img_h - 1)
    sys_col = sys.reshape(no, 1)
    sys_row = sys.reshape(1, no)
    params = jnp.stack([jnp.float32(img_h - 1), jnp.float32(img_w - 1),
                        jnp.float32(img_h), jnp.float32(0.0)]).reshape(1, 4)

    outm, outi = _run(predsT, gt_scal, gdelC, gt_cols, gdelw, sys_col,
                      sys_row, params, b=b, p=p, t=t, no=no)
    assigned = outm.reshape(b, p).astype(bool)
    matched = outi.reshape(b, p).astype(jax.dtypes.canonicalize_dtype(np.int64))
    return assigned, matched
